# Initial kernel scaffold; baseline (speedup 1.0000x reference)
import jax
import jax.numpy as jnp
from jax.experimental import pallas as pl


def kernel(user_emb, item_emb, edge_index, users):
    raise NotImplementedError("write your pallas kernel here")


# speedup vs baseline: 8.7330x; 8.7330x over previous
import functools

import jax
import jax.numpy as jnp
from jax import lax
from jax.experimental import pallas as pl
from jax.experimental.pallas import tpu as pltpu
from jax.experimental.pallas import tpu_sc as plsc

NU = 25000
NI = 25000
D = 64
NUP = 25088
NROWS = 2 * NUP
NTILES = 16
PER_TILE_ROWS = NUP // NTILES
CH = 128
CHUNKS_PER_TILE = 400
SUPER = 16
N_SUPER = CHUNKS_PER_TILE // SUPER
EPC = NTILES * CHUNKS_PER_TILE * CH
NB = 2048


def _sc_layer_body(y_hbm, rows_hbm, cols_hbm, dinv_hbm, out_hbm,
                   acc, rows_blk, cols_blk, gbuf, dvt):
    c = lax.axis_index("c")
    s = lax.axis_index("s")

    def _zero_g(r, _):
        for v in range(4):
            gbuf[r, pl.ds(v * 16, 16)] = jnp.zeros((16,), jnp.float32)
        return _
    lax.fori_loop(0, CH, _zero_g, None)
    row0 = s * PER_TILE_ROWS
    for off, n in [(j * CH, CH) for j in range(PER_TILE_ROWS // CH)] + \
                  [(PER_TILE_ROWS - PER_TILE_ROWS % CH, PER_TILE_ROWS % CH)]:
        if n:
            pltpu.sync_copy(gbuf.at[pl.ds(0, n), :], acc.at[pl.ds(row0 + off, n), :])
    pltpu.sync_copy(dinv_hbm.at[pl.ds(c * NUP + row0, PER_TILE_ROWS)],
                    dvt.at[pl.ds(0, PER_TILE_ROWS)])
    plsc.subcore_barrier()

    cbase = s * CHUNKS_PER_TILE

    def _super(i, _):
        pltpu.sync_copy(rows_hbm.at[c, pl.ds(cbase + i * SUPER, SUPER), :], rows_blk)
        pltpu.sync_copy(cols_hbm.at[c, pl.ds(cbase + i * SUPER, SUPER), :], cols_blk)
        for j in range(SUPER):
            pltpu.sync_copy(y_hbm.at[cols_blk.at[j]], gbuf)
            pltpu.sync_copy(gbuf, acc.at[rows_blk.at[j]], add=True)
        return _
    lax.fori_loop(0, N_SUPER, _super, None)
    plsc.subcore_barrier()

    for off, n in [(j * CH, CH) for j in range(PER_TILE_ROWS // CH)] + \
                  [(PER_TILE_ROWS - PER_TILE_ROWS % CH, PER_TILE_ROWS % CH)]:
        if not n:
            continue
        pltpu.sync_copy(acc.at[pl.ds(row0 + off, n), :], gbuf.at[pl.ds(0, n), :])

        def _scale(r, _):
            d = dvt[pl.ds(off + r, 16)][0]
            for v in range(4):
                sl = pl.ds(v * 16, 16)
                gbuf[r, sl] = gbuf[r, sl] * d
            return _
        lax.fori_loop(0, n, _scale, None)
        pltpu.sync_copy(gbuf.at[pl.ds(0, n), :],
                        out_hbm.at[pl.ds(c * NUP + row0 + off, n), :])


@functools.partial(jax.jit, static_argnums=())
def _sc_layer(y, rows_st, cols_st, d_inv):
    mesh = plsc.VectorSubcoreMesh(core_axis_name="c", subcore_axis_name="s")
    f = pl.kernel(
        _sc_layer_body,
        out_type=jax.ShapeDtypeStruct((NROWS, D), jnp.float32),
        mesh=mesh,
        compiler_params=pltpu.CompilerParams(use_tc_tiling_on_sc=False),
        scratch_types=[
            pltpu.VMEM_SHARED((NUP, D), jnp.float32),
            pltpu.VMEM((SUPER, CH), jnp.int32),
            pltpu.VMEM((SUPER, CH), jnp.int32),
            pltpu.VMEM((CH, D), jnp.float32),
            pltpu.VMEM((PER_TILE_ROWS + 16,), jnp.float32),
        ],
    )
    return f(y, rows_st, cols_st, d_inv)


def _tc_rating_body(ue0, uy, dsqu, ie0, iy1, iy2, iy3, dsqi, out):
    um = (ue0[...] + dsqu[0, :][:, None] * uy[...]) * 0.25
    im = (ie0[...] + dsqi[0, :][:, None] * (iy1[...] + iy2[...] + iy3[...])) * 0.25
    logits = lax.dot_general(um, im, (((1,), (1,)), ((), ())),
                             preferred_element_type=jnp.float32)
    out[...] = 1.0 / (1.0 + jnp.exp(-logits))


def _tc_rating(ue0, uy, dsqu, ie0, iy1, iy2, iy3, dsqi):
    q = ue0.shape[0]
    grid = (NI + NB - 1) // NB
    return pl.pallas_call(
        _tc_rating_body,
        grid=(grid,),
        in_specs=[
            pl.BlockSpec((q, D), lambda j: (0, 0)),
            pl.BlockSpec((q, D), lambda j: (0, 0)),
            pl.BlockSpec((1, q), lambda j: (0, 0)),
            pl.BlockSpec((NB, D), lambda j: (j, 0)),
            pl.BlockSpec((NB, D), lambda j: (j, 0)),
            pl.BlockSpec((NB, D), lambda j: (j, 0)),
            pl.BlockSpec((NB, D), lambda j: (j, 0)),
            pl.BlockSpec((1, NB), lambda j: (0, j)),
        ],
        out_specs=pl.BlockSpec((q, NB), lambda j: (0, j)),
        out_shape=jax.ShapeDtypeStruct((q, NI), jnp.float32),
    )(ue0, uy, dsqu, ie0, iy1, iy2, iy3, dsqi)


def kernel(user_emb, item_emb, edge_index, users):
    src = edge_index[0]
    dst = edge_index[1]
    e = src.shape[0]

    deg_u = jnp.zeros((NU,), jnp.float32).at[src].add(1.0)
    deg_i = jnp.zeros((NI,), jnp.float32).at[dst].add(1.0)

    def _padded(x_u, x_i):
        z = jnp.zeros((NROWS,), jnp.float32)
        return z.at[:NU].set(x_u).at[NUP:NUP + NI].set(x_i)

    d_isr_u = jnp.where(deg_u > 0, lax.rsqrt(jnp.maximum(deg_u, 1.0)), 0.0)
    d_isr_i = jnp.where(deg_i > 0, lax.rsqrt(jnp.maximum(deg_i, 1.0)), 0.0)
    d_inv = _padded(d_isr_u * d_isr_u, d_isr_i * d_isr_i)
    dsq_u = jnp.sqrt(deg_u)
    dsq_i = jnp.sqrt(deg_i)

    y0 = jnp.zeros((NROWS, D), jnp.float32)
    y0 = y0.at[:NU].set(user_emb * d_isr_u[:, None])
    y0 = y0.at[NUP:NUP + NI].set(item_emb * d_isr_i[:, None])

    pad = EPC - e
    pad_r = jnp.full((pad,), NU, jnp.int32)
    pad_c = jnp.zeros((pad,), jnp.int32)
    rows0 = jnp.concatenate([src, pad_r]).reshape(NTILES * CHUNKS_PER_TILE, CH)
    cols0 = jnp.concatenate([dst + NUP, pad_c]).reshape(NTILES * CHUNKS_PER_TILE, CH)
    rows1 = jnp.concatenate([dst, pad_r]).reshape(NTILES * CHUNKS_PER_TILE, CH)
    cols1 = jnp.concatenate([src, pad_c]).reshape(NTILES * CHUNKS_PER_TILE, CH)
    rows_st = jnp.stack([rows0, rows1])
    cols_st = jnp.stack([cols0, cols1])

    y1 = _sc_layer(y0, rows_st, cols_st, d_inv)
    y2 = _sc_layer(y1, rows_st, cols_st, d_inv)
    y3 = _sc_layer(y2, rows_st, cols_st, d_inv)

    ue0 = user_emb[users]
    uy = y1[users] + y2[users] + y3[users]
    dsqu = dsq_u[users][None, :]
    iy1 = y1[NUP:NUP + NI]
    iy2 = y2[NUP:NUP + NI]
    iy3 = y3[NUP:NUP + NI]
    return _tc_rating(ue0, uy, dsqu, item_emb, iy1, iy2, iy3, dsq_i[None, :])


# speedup vs baseline: 9.1352x; 1.0461x over previous
import functools

import jax
import jax.numpy as jnp
from jax import lax
from jax.experimental import pallas as pl
from jax.experimental.pallas import tpu as pltpu
from jax.experimental.pallas import tpu_sc as plsc

NU = 25000
NI = 25000
D = 64
NUP = 25088
NROWS = 2 * NUP
NTILES = 16
PER_TILE_ROWS = NUP // NTILES
CH = 128
CHUNKS_PER_TILE = 400
SUPER = 16
N_SUPER = CHUNKS_PER_TILE // SUPER
EPC = NTILES * CHUNKS_PER_TILE * CH
NB = 2048


def _sc_layer_body(y_hbm, rows_hbm, cols_hbm, dinv_hbm, out_hbm,
                   acc, rows_blk, cols_blk, gbuf, gbuf1, dvt, gsem0, gsem1):
    c = lax.axis_index("c")
    s = lax.axis_index("s")

    def _zero_g(r, _):
        for v in range(4):
            gbuf[r, pl.ds(v * 16, 16)] = jnp.zeros((16,), jnp.float32)
        return _
    lax.fori_loop(0, CH, _zero_g, None)
    row0 = s * PER_TILE_ROWS
    for off, n in [(j * CH, CH) for j in range(PER_TILE_ROWS // CH)] + \
                  [(PER_TILE_ROWS - PER_TILE_ROWS % CH, PER_TILE_ROWS % CH)]:
        if n:
            pltpu.sync_copy(gbuf.at[pl.ds(0, n), :], acc.at[pl.ds(row0 + off, n), :])
    pltpu.sync_copy(dinv_hbm.at[pl.ds(c * NUP + row0, PER_TILE_ROWS)],
                    dvt.at[pl.ds(0, PER_TILE_ROWS)])
    plsc.subcore_barrier()

    cbase = s * CHUNKS_PER_TILE

    bufs = (gbuf, gbuf1)
    sems = (gsem0, gsem1)

    def _super(i, _):
        pltpu.sync_copy(rows_hbm.at[c, pl.ds(cbase + i * SUPER, SUPER), :], rows_blk)
        pltpu.sync_copy(cols_hbm.at[c, pl.ds(cbase + i * SUPER, SUPER), :], cols_blk)
        d = pltpu.async_copy(y_hbm.at[cols_blk.at[0]], bufs[0], sems[0])
        for j in range(SUPER):
            d.wait()
            if j + 1 < SUPER:
                d = pltpu.async_copy(y_hbm.at[cols_blk.at[j + 1]],
                                     bufs[(j + 1) % 2], sems[(j + 1) % 2])
            pltpu.sync_copy(bufs[j % 2], acc.at[rows_blk.at[j]], add=True)
        return _
    lax.fori_loop(0, N_SUPER, _super, None)
    plsc.subcore_barrier()

    for off, n in [(j * CH, CH) for j in range(PER_TILE_ROWS // CH)] + \
                  [(PER_TILE_ROWS - PER_TILE_ROWS % CH, PER_TILE_ROWS % CH)]:
        if not n:
            continue
        pltpu.sync_copy(acc.at[pl.ds(row0 + off, n), :], gbuf.at[pl.ds(0, n), :])

        def _scale(r, _):
            d = dvt[pl.ds(off + r, 16)][0]
            for v in range(4):
                sl = pl.ds(v * 16, 16)
                gbuf[r, sl] = gbuf[r, sl] * d
            return _
        lax.fori_loop(0, n, _scale, None)
        pltpu.sync_copy(gbuf.at[pl.ds(0, n), :],
                        out_hbm.at[pl.ds(c * NUP + row0 + off, n), :])


@functools.partial(jax.jit, static_argnums=())
def _sc_layer(y, rows_st, cols_st, d_inv):
    mesh = plsc.VectorSubcoreMesh(core_axis_name="c", subcore_axis_name="s")
    f = pl.kernel(
        _sc_layer_body,
        out_type=jax.ShapeDtypeStruct((NROWS, D), jnp.float32),
        mesh=mesh,
        compiler_params=pltpu.CompilerParams(use_tc_tiling_on_sc=False),
        scratch_types=[
            pltpu.VMEM_SHARED((NUP, D), jnp.float32),
            pltpu.VMEM((SUPER, CH), jnp.int32),
            pltpu.VMEM((SUPER, CH), jnp.int32),
            pltpu.VMEM((CH, D), jnp.float32),
            pltpu.VMEM((CH, D), jnp.float32),
            pltpu.VMEM((PER_TILE_ROWS + 16,), jnp.float32),
            pltpu.SemaphoreType.DMA,
            pltpu.SemaphoreType.DMA,
        ],
    )
    return f(y, rows_st, cols_st, d_inv)


def _tc_rating_body(ue0, uy, dsqu, ie0, iy1, iy2, iy3, dsqi, out):
    um = (ue0[...] + dsqu[0, :][:, None] * uy[...]) * 0.25
    im = (ie0[...] + dsqi[0, :][:, None] * (iy1[...] + iy2[...] + iy3[...])) * 0.25
    logits = lax.dot_general(um, im, (((1,), (1,)), ((), ())),
                             preferred_element_type=jnp.float32)
    out[...] = 1.0 / (1.0 + jnp.exp(-logits))


def _tc_rating(ue0, uy, dsqu, ie0, iy1, iy2, iy3, dsqi):
    q = ue0.shape[0]
    grid = (NI + NB - 1) // NB
    return pl.pallas_call(
        _tc_rating_body,
        grid=(grid,),
        in_specs=[
            pl.BlockSpec((q, D), lambda j: (0, 0)),
            pl.BlockSpec((q, D), lambda j: (0, 0)),
            pl.BlockSpec((1, q), lambda j: (0, 0)),
            pl.BlockSpec((NB, D), lambda j: (j, 0)),
            pl.BlockSpec((NB, D), lambda j: (j, 0)),
            pl.BlockSpec((NB, D), lambda j: (j, 0)),
            pl.BlockSpec((NB, D), lambda j: (j, 0)),
            pl.BlockSpec((1, NB), lambda j: (0, j)),
        ],
        out_specs=pl.BlockSpec((q, NB), lambda j: (0, j)),
        out_shape=jax.ShapeDtypeStruct((q, NI), jnp.float32),
    )(ue0, uy, dsqu, ie0, iy1, iy2, iy3, dsqi)


def kernel(user_emb, item_emb, edge_index, users):
    src = edge_index[0]
    dst = edge_index[1]
    e = src.shape[0]

    deg_u = jnp.zeros((NU,), jnp.float32).at[src].add(1.0)
    deg_i = jnp.zeros((NI,), jnp.float32).at[dst].add(1.0)

    def _padded(x_u, x_i):
        z = jnp.zeros((NROWS,), jnp.float32)
        return z.at[:NU].set(x_u).at[NUP:NUP + NI].set(x_i)

    d_isr_u = jnp.where(deg_u > 0, lax.rsqrt(jnp.maximum(deg_u, 1.0)), 0.0)
    d_isr_i = jnp.where(deg_i > 0, lax.rsqrt(jnp.maximum(deg_i, 1.0)), 0.0)
    d_inv = _padded(d_isr_u * d_isr_u, d_isr_i * d_isr_i)
    dsq_u = jnp.sqrt(deg_u)
    dsq_i = jnp.sqrt(deg_i)

    y0 = jnp.zeros((NROWS, D), jnp.float32)
    y0 = y0.at[:NU].set(user_emb * d_isr_u[:, None])
    y0 = y0.at[NUP:NUP + NI].set(item_emb * d_isr_i[:, None])

    pad = EPC - e
    pad_r = jnp.full((pad,), NU, jnp.int32)
    pad_c = jnp.zeros((pad,), jnp.int32)
    rows0 = jnp.concatenate([src, pad_r]).reshape(NTILES * CHUNKS_PER_TILE, CH)
    cols0 = jnp.concatenate([dst + NUP, pad_c]).reshape(NTILES * CHUNKS_PER_TILE, CH)
    rows1 = jnp.concatenate([dst, pad_r]).reshape(NTILES * CHUNKS_PER_TILE, CH)
    cols1 = jnp.concatenate([src, pad_c]).reshape(NTILES * CHUNKS_PER_TILE, CH)
    rows_st = jnp.stack([rows0, rows1])
    cols_st = jnp.stack([cols0, cols1])

    y1 = _sc_layer(y0, rows_st, cols_st, d_inv)
    y2 = _sc_layer(y1, rows_st, cols_st, d_inv)
    y3 = _sc_layer(y2, rows_st, cols_st, d_inv)

    ue0 = user_emb[users]
    uy = y1[users] + y2[users] + y3[users]
    dsqu = dsq_u[users][None, :]
    iy1 = y1[NUP:NUP + NI]
    iy2 = y2[NUP:NUP + NI]
    iy3 = y3[NUP:NUP + NI]
    return _tc_rating(ue0, uy, dsqu, item_emb, iy1, iy2, iy3, dsq_i[None, :])


# speedup vs baseline: 9.4336x; 1.0327x over previous
import functools

import jax
import jax.numpy as jnp
from jax import lax
from jax.experimental import pallas as pl
from jax.experimental.pallas import tpu as pltpu
from jax.experimental.pallas import tpu_sc as plsc

NU = 25000
NI = 25000
D = 64
NUP = 25088
NROWS = 2 * NUP
NTILES = 16
PER_TILE_ROWS = NUP // NTILES
CH = 80
CHUNKS_PER_TILE = 640
SUPER = 16
N_SUPER = CHUNKS_PER_TILE // SUPER
EPC = NTILES * CHUNKS_PER_TILE * CH
NB = 2048


def _sc_layer_body(y_hbm, rows_hbm, cols_hbm, dinv_hbm, out_hbm,
                   acc, rows_blk, cols_blk, gbuf, gbuf1, gbuf2, gbuf3, dvt,
                   gsems, ssems):
    c = lax.axis_index("c")
    s = lax.axis_index("s")

    def _zero_g(r, _):
        for v in range(4):
            gbuf[r, pl.ds(v * 16, 16)] = jnp.zeros((16,), jnp.float32)
        return _
    lax.fori_loop(0, CH, _zero_g, None)
    row0 = s * PER_TILE_ROWS
    for off, n in [(j * CH, CH) for j in range(PER_TILE_ROWS // CH)] + \
                  [(PER_TILE_ROWS - PER_TILE_ROWS % CH, PER_TILE_ROWS % CH)]:
        if n:
            pltpu.sync_copy(gbuf.at[pl.ds(0, n), :], acc.at[pl.ds(row0 + off, n), :])
    pltpu.sync_copy(dinv_hbm.at[pl.ds(c * NUP + row0, PER_TILE_ROWS)],
                    dvt.at[pl.ds(0, PER_TILE_ROWS)])
    plsc.subcore_barrier()

    cbase = s * CHUNKS_PER_TILE

    bufs = (gbuf, gbuf1, gbuf2, gbuf3)
    NBUF = 4
    LOOK = 3

    def _super(i, _):
        pltpu.sync_copy(rows_hbm.at[c, pl.ds(cbase + i * SUPER, SUPER), :], rows_blk)
        pltpu.sync_copy(cols_hbm.at[c, pl.ds(cbase + i * SUPER, SUPER), :], cols_blk)
        dg = [None] * SUPER
        dsc = [None] * SUPER
        for j in range(LOOK):
            dg[j] = pltpu.async_copy(y_hbm.at[cols_blk.at[j]], bufs[j],
                                     gsems.at[j])
        for j in range(SUPER):
            b = j % NBUF
            dg[j].wait()
            dsc[j] = pltpu.async_copy(bufs[b], acc.at[rows_blk.at[j]],
                                      ssems.at[b], add=True)
            nj = j + LOOK
            if nj < SUPER:
                nb = nj % NBUF
                if nj >= NBUF:
                    dsc[nj - NBUF].wait()
                dg[nj] = pltpu.async_copy(y_hbm.at[cols_blk.at[nj]], bufs[nb],
                                          gsems.at[nb])
        for j in range(SUPER - NBUF, SUPER):
            dsc[j].wait()
        return _
    lax.fori_loop(0, N_SUPER, _super, None)
    plsc.subcore_barrier()

    for off, n in [(j * CH, CH) for j in range(PER_TILE_ROWS // CH)] + \
                  [(PER_TILE_ROWS - PER_TILE_ROWS % CH, PER_TILE_ROWS % CH)]:
        if not n:
            continue
        pltpu.sync_copy(acc.at[pl.ds(row0 + off, n), :], gbuf.at[pl.ds(0, n), :])

        def _scale(r, _):
            d = dvt[pl.ds(off + r, 16)][0]
            for v in range(4):
                sl = pl.ds(v * 16, 16)
                gbuf[r, sl] = gbuf[r, sl] * d
            return _
        lax.fori_loop(0, n, _scale, None)
        pltpu.sync_copy(gbuf.at[pl.ds(0, n), :],
                        out_hbm.at[pl.ds(c * NUP + row0 + off, n), :])


@functools.partial(jax.jit, static_argnums=())
def _sc_layer(y, rows_st, cols_st, d_inv):
    mesh = plsc.VectorSubcoreMesh(core_axis_name="c", subcore_axis_name="s")
    f = pl.kernel(
        _sc_layer_body,
        out_type=jax.ShapeDtypeStruct((NROWS, D), jnp.float32),
        mesh=mesh,
        compiler_params=pltpu.CompilerParams(use_tc_tiling_on_sc=False),
        scratch_types=[
            pltpu.VMEM_SHARED((NUP, D), jnp.float32),
            pltpu.VMEM((SUPER, CH), jnp.int32),
            pltpu.VMEM((SUPER, CH), jnp.int32),
            pltpu.VMEM((CH, D), jnp.float32),
            pltpu.VMEM((CH, D), jnp.float32),
            pltpu.VMEM((CH, D), jnp.float32),
            pltpu.VMEM((CH, D), jnp.float32),
            pltpu.VMEM((PER_TILE_ROWS + 16,), jnp.float32),
            pltpu.SemaphoreType.DMA((4,)),
            pltpu.SemaphoreType.DMA((4,)),
        ],
    )
    return f(y, rows_st, cols_st, d_inv)


def _tc_rating_body(ue0, uy, dsqu, ie0, iy1, iy2, iy3, dsqi, out):
    um = (ue0[...] + dsqu[0, :][:, None] * uy[...]) * 0.25
    im = (ie0[...] + dsqi[0, :][:, None] * (iy1[...] + iy2[...] + iy3[...])) * 0.25
    logits = lax.dot_general(um, im, (((1,), (1,)), ((), ())),
                             preferred_element_type=jnp.float32)
    out[...] = 1.0 / (1.0 + jnp.exp(-logits))


def _tc_rating(ue0, uy, dsqu, ie0, iy1, iy2, iy3, dsqi):
    q = ue0.shape[0]
    grid = (NI + NB - 1) // NB
    return pl.pallas_call(
        _tc_rating_body,
        grid=(grid,),
        in_specs=[
            pl.BlockSpec((q, D), lambda j: (0, 0)),
            pl.BlockSpec((q, D), lambda j: (0, 0)),
            pl.BlockSpec((1, q), lambda j: (0, 0)),
            pl.BlockSpec((NB, D), lambda j: (j, 0)),
            pl.BlockSpec((NB, D), lambda j: (j, 0)),
            pl.BlockSpec((NB, D), lambda j: (j, 0)),
            pl.BlockSpec((NB, D), lambda j: (j, 0)),
            pl.BlockSpec((1, NB), lambda j: (0, j)),
        ],
        out_specs=pl.BlockSpec((q, NB), lambda j: (0, j)),
        out_shape=jax.ShapeDtypeStruct((q, NI), jnp.float32),
    )(ue0, uy, dsqu, ie0, iy1, iy2, iy3, dsqi)


def kernel(user_emb, item_emb, edge_index, users):
    src = edge_index[0]
    dst = edge_index[1]
    e = src.shape[0]

    deg_u = jnp.zeros((NU,), jnp.float32).at[src].add(1.0)
    deg_i = jnp.zeros((NI,), jnp.float32).at[dst].add(1.0)

    def _padded(x_u, x_i):
        z = jnp.zeros((NROWS,), jnp.float32)
        return z.at[:NU].set(x_u).at[NUP:NUP + NI].set(x_i)

    d_isr_u = jnp.where(deg_u > 0, lax.rsqrt(jnp.maximum(deg_u, 1.0)), 0.0)
    d_isr_i = jnp.where(deg_i > 0, lax.rsqrt(jnp.maximum(deg_i, 1.0)), 0.0)
    d_inv = _padded(d_isr_u * d_isr_u, d_isr_i * d_isr_i)
    dsq_u = jnp.sqrt(deg_u)
    dsq_i = jnp.sqrt(deg_i)

    y0 = jnp.zeros((NROWS, D), jnp.float32)
    y0 = y0.at[:NU].set(user_emb * d_isr_u[:, None])
    y0 = y0.at[NUP:NUP + NI].set(item_emb * d_isr_i[:, None])

    pad = EPC - e
    pad_r = jnp.full((pad,), NU, jnp.int32)
    pad_c = jnp.zeros((pad,), jnp.int32)
    rows0 = jnp.concatenate([src, pad_r]).reshape(NTILES * CHUNKS_PER_TILE, CH)
    cols0 = jnp.concatenate([dst + NUP, pad_c]).reshape(NTILES * CHUNKS_PER_TILE, CH)
    rows1 = jnp.concatenate([dst, pad_r]).reshape(NTILES * CHUNKS_PER_TILE, CH)
    cols1 = jnp.concatenate([src, pad_c]).reshape(NTILES * CHUNKS_PER_TILE, CH)
    rows_st = jnp.stack([rows0, rows1])
    cols_st = jnp.stack([cols0, cols1])

    y1 = _sc_layer(y0, rows_st, cols_st, d_inv)
    y2 = _sc_layer(y1, rows_st, cols_st, d_inv)
    y3 = _sc_layer(y2, rows_st, cols_st, d_inv)

    ue0 = user_emb[users]
    uy = y1[users] + y2[users] + y3[users]
    dsqu = dsq_u[users][None, :]
    iy1 = y1[NUP:NUP + NI]
    iy2 = y2[NUP:NUP + NI]
    iy3 = y3[NUP:NUP + NI]
    return _tc_rating(ue0, uy, dsqu, item_emb, iy1, iy2, iy3, dsq_i[None, :])
